# baseline (speedup 1.0000x reference)
import functools

import jax
import jax.numpy as jnp
from jax import lax
from jax.experimental import pallas as pl
from jax.experimental.pallas import tpu as pltpu
from jax.experimental.pallas import tpu_sc as plsc

_VOCAB = 32
_NEG = -1e9
_POS = 1e9


def kernel(hidden, next_ids):
    del hidden
    B, T = next_ids.shape
    N = B * T
    ids = next_ids.reshape(N).astype(jnp.int32)
    NW_CHUNKS = 8

    info = plsc.get_sparse_core_info()
    NC, NS, L = info.num_cores, info.num_subcores, info.num_lanes
    NW = NC * NS
    nper = N // NW

    mesh = plsc.VectorSubcoreMesh(core_axis_name="c", subcore_axis_name="s")

    chunk = (nper * _VOCAB) // NW_CHUNKS
    g_per_chunk = nper // (NW_CHUNKS * L)

    @functools.partial(
        pl.kernel,
        mesh=mesh,
        out_type=jax.ShapeDtypeStruct((N * _VOCAB,), jnp.float32),
        scratch_types=[
            pltpu.VMEM((nper,), jnp.int32),
            pltpu.VMEM((nper * _VOCAB,), jnp.float32),
            pltpu.SemaphoreType.DMA,
            pltpu.SemaphoreType.DMA,
        ],
        compiler_params=pltpu.CompilerParams(needs_layout_passes=False),
    )
    def sc_onehot(ids_hbm, out_hbm, idx_v, buf, sem_ids, sem_out):
        wid = lax.axis_index("s") * NC + lax.axis_index("c")
        base = wid * nper

        id_cp = pltpu.async_copy(ids_hbm.at[pl.ds(base, nper)], idx_v,
                                 sem_ids)

        neg = jnp.full((L,), _NEG, jnp.float32)
        lane = lax.iota(jnp.int32, L)
        pos = jnp.full((L,), _POS, jnp.float32)

        def init_body(i, c):
            for u in range(16):
                buf[pl.ds((i * 16 + u) * L, L)] = neg
            return c

        def scat_body(g, c):
            tok = g * L
            v = idx_v[pl.ds(tok, L)]
            tgt = (v + 1) & (_VOCAB - 1)
            flat = (lane + tok) * _VOCAB + tgt
            plsc.store_scatter(buf, [flat], pos)
            return c

        i_per_chunk = chunk // (16 * L)
        out_cps = []

        def ship(k):
            lax.fori_loop(k * g_per_chunk, (k + 1) * g_per_chunk,
                          scat_body, 0)
            out_cps.append(
                pltpu.async_copy(
                    buf.at[pl.ds(k * chunk, chunk)],
                    out_hbm.at[pl.ds(base * _VOCAB + k * chunk, chunk)],
                    sem_out,
                )
            )

        lax.fori_loop(0, i_per_chunk, init_body, 0)
        id_cp.wait()
        for k in range(1, NW_CHUNKS):
            lax.fori_loop(k * i_per_chunk, (k + 1) * i_per_chunk,
                          init_body, 0)
            ship(k - 1)
        ship(NW_CHUNKS - 1)
        for cp in out_cps:
            cp.wait()

    out = sc_onehot(ids)
    return out.reshape(B, T, _VOCAB)


# speedup vs baseline: 1.0103x; 1.0103x over previous
import functools

import jax
import jax.numpy as jnp
from jax import lax
from jax.experimental import pallas as pl
from jax.experimental.pallas import tpu as pltpu
from jax.experimental.pallas import tpu_sc as plsc

_VOCAB = 32
_NEG = -1e9
_POS = 1e9


def kernel(hidden, next_ids):
    del hidden
    B, T = next_ids.shape
    N = B * T
    ids = next_ids.reshape(N).astype(jnp.int32)
    NW_CHUNKS = 4

    info = plsc.get_sparse_core_info()
    NC, NS, L = info.num_cores, info.num_subcores, info.num_lanes
    NW = NC * NS
    nper = N // NW
    t_chunk = nper // NW_CHUNKS
    g_per_chunk = t_chunk // L

    mesh = plsc.VectorSubcoreMesh(core_axis_name="c", subcore_axis_name="s")

    @functools.partial(
        pl.kernel,
        mesh=mesh,
        out_type=jax.ShapeDtypeStruct((B, T, _VOCAB), jnp.float32),
        scratch_types=[
            pltpu.VMEM((nper,), jnp.int32),
            pltpu.VMEM((nper, _VOCAB), jnp.float32),
            pltpu.SemaphoreType.DMA,
            pltpu.SemaphoreType.DMA,
        ],
        compiler_params=pltpu.CompilerParams(
            needs_layout_passes=False, use_tc_tiling_on_sc=False
        ),
    )
    def sc_onehot(ids_hbm, out_hbm, idx_v, buf, sem_ids, sem_out):
        wid = lax.axis_index("s") * NC + lax.axis_index("c")
        base = wid * nper
        b = base // T
        t0 = base % T

        id_cp = pltpu.async_copy(ids_hbm.at[pl.ds(base, nper)], idx_v,
                                 sem_ids)

        neg = jnp.full((L,), _NEG, jnp.float32)
        lane = lax.iota(jnp.int32, L)
        pos = jnp.full((L,), _POS, jnp.float32)

        def init_body(i, c):
            for u in range(8):
                row = i * 8 + u
                buf[row, pl.ds(0, L)] = neg
                buf[row, pl.ds(L, L)] = neg
            return c

        def scat_body(g, c):
            tok = g * L
            v = idx_v[pl.ds(tok, L)]
            tgt = (v + 1) & (_VOCAB - 1)
            plsc.store_scatter(buf, [lane + tok, tgt], pos)
            return c

        i_per_chunk = t_chunk // 8
        out_cps = []

        def ship(k):
            lax.fori_loop(k * g_per_chunk, (k + 1) * g_per_chunk,
                          scat_body, 0)
            out_cps.append(
                pltpu.async_copy(
                    buf.at[pl.ds(k * t_chunk, t_chunk), :],
                    out_hbm.at[b, pl.ds(t0 + k * t_chunk, t_chunk), :],
                    sem_out,
                )
            )

        lax.fori_loop(0, i_per_chunk, init_body, 0)
        id_cp.wait()
        for k in range(1, NW_CHUNKS):
            lax.fori_loop(k * i_per_chunk, (k + 1) * i_per_chunk,
                          init_body, 0)
            ship(k - 1)
        ship(NW_CHUNKS - 1)
        for cp in out_cps:
            cp.wait()

    return sc_onehot(ids)


# speedup vs baseline: 1.2471x; 1.2343x over previous
import functools

import jax
import jax.numpy as jnp
from jax import lax
from jax.experimental import pallas as pl
from jax.experimental.pallas import tpu as pltpu
from jax.experimental.pallas import tpu_sc as plsc

_VOCAB = 32
_NEG = -1e9
_POS = 1e9


def kernel(hidden, next_ids):
    del hidden
    B, T = next_ids.shape
    N = B * T
    ids = next_ids.reshape(N).astype(jnp.int32)
    NW_CHUNKS = 4

    info = plsc.get_sparse_core_info()
    NC, NS, L = info.num_cores, info.num_subcores, info.num_lanes
    NW = NC * NS
    nper = N // NW
    t_chunk = nper // NW_CHUNKS
    g_per_chunk = t_chunk // L

    mesh = plsc.VectorSubcoreMesh(core_axis_name="c", subcore_axis_name="s")

    @functools.partial(
        pl.kernel,
        mesh=mesh,
        out_type=jax.ShapeDtypeStruct((B, T, _VOCAB), jnp.float32),
        scratch_types=[
            pltpu.VMEM((nper,), jnp.int32),
            pltpu.VMEM((2, t_chunk, _VOCAB), jnp.float32),
            pltpu.SemaphoreType.DMA,
            pltpu.SemaphoreType.DMA,
        ],
        compiler_params=pltpu.CompilerParams(needs_layout_passes=False),
    )
    def sc_onehot(ids_hbm, out_hbm, idx_v, buf, sem_ids, sem_out):
        wid = lax.axis_index("s") * NC + lax.axis_index("c")
        base = wid * nper
        b = base // T
        t0 = base % T

        id_cp = pltpu.async_copy(ids_hbm.at[pl.ds(base, nper)], idx_v,
                                 sem_ids)

        neg = jnp.full((L,), _NEG, jnp.float32)
        lane = lax.iota(jnp.int32, L)
        pos = jnp.full((L,), _POS, jnp.float32)

        def fill(p):
            def body(i, c):
                for u in range(8):
                    row = i * 8 + u
                    buf[p, row, pl.ds(0, L)] = neg
                    buf[p, row, pl.ds(L, L)] = neg
                return c

            lax.fori_loop(0, t_chunk // 8, body, 0)

        def scat(k, p):
            def body(g, c):
                tok = g * L
                v = idx_v[pl.ds(tok, L)]
                tgt = (v + 1) & (_VOCAB - 1)
                plsc.store_scatter(buf.at[p], [lane + tok - k * t_chunk,
                                               tgt], pos)
                return c

            lax.fori_loop(k * g_per_chunk, (k + 1) * g_per_chunk, body, 0)

        out_cps = [None] * NW_CHUNKS
        fill(0)
        id_cp.wait()
        for k in range(NW_CHUNKS):
            p = k & 1
            scat(k, p)
            out_cps[k] = pltpu.async_copy(
                buf.at[p],
                out_hbm.at[b, pl.ds(t0 + k * t_chunk, t_chunk), :],
                sem_out,
            )
            if k + 1 < NW_CHUNKS:
                if k >= 1:
                    out_cps[k - 1].wait()
                fill(1 - p)
        out_cps[NW_CHUNKS - 2].wait()
        out_cps[NW_CHUNKS - 1].wait()

    return sc_onehot(ids)


# speedup vs baseline: 2.0163x; 1.6168x over previous
import functools

import jax
import jax.numpy as jnp
from jax import lax
from jax.experimental import pallas as pl
from jax.experimental.pallas import tpu as pltpu
from jax.experimental.pallas import tpu_sc as plsc

_VOCAB = 32
_NEG = -1e9
_POS = 1e9


def kernel(hidden, next_ids):
    del hidden
    B, T = next_ids.shape
    N = B * T
    ids = next_ids.reshape(N).astype(jnp.int32)
    NW_CHUNKS = 4

    info = plsc.get_sparse_core_info()
    NC, NS, L = info.num_cores, info.num_subcores, info.num_lanes
    NW = NC * NS
    nper = N // NW
    t_chunk = nper // NW_CHUNKS
    g_per_chunk = t_chunk // L

    mesh = plsc.VectorSubcoreMesh(core_axis_name="c", subcore_axis_name="s")

    @functools.partial(
        pl.kernel,
        mesh=mesh,
        out_type=jax.ShapeDtypeStruct((B, _VOCAB, T), jnp.float32),
        scratch_types=[
            pltpu.VMEM((nper,), jnp.int32),
            pltpu.VMEM((_VOCAB, nper), jnp.float32),
            pltpu.SemaphoreType.DMA,
            pltpu.SemaphoreType.DMA,
        ],
        compiler_params=pltpu.CompilerParams(needs_layout_passes=False),
    )
    def sc_onehot(ids_hbm, out_hbm, idx_v, buf, sem_ids, sem_out):
        wid = lax.axis_index("s") * NC + lax.axis_index("c")
        base = wid * nper
        b = base // T
        t0 = base % T

        id_cp = pltpu.async_copy(ids_hbm.at[pl.ds(base, nper)], idx_v,
                                 sem_ids)

        neg = jnp.full((L,), _NEG, jnp.float32)
        lane = lax.iota(jnp.int32, L)
        pos = jnp.full((L,), _POS, jnp.float32)

        def fill(k):
            def body(i, c):
                col = k * t_chunk + i * L
                for v in range(_VOCAB):
                    buf[v, pl.ds(col, L)] = neg
                return c

            lax.fori_loop(0, t_chunk // L, body, 0)

        def scat(k):
            def body(g, c):
                tok = g * L
                v = idx_v[pl.ds(tok, L)]
                tgt = (v + 1) & (_VOCAB - 1)
                plsc.store_scatter(buf, [tgt, lane + tok], pos)
                return c

            lax.fori_loop(k * g_per_chunk, (k + 1) * g_per_chunk, body, 0)

        out_cps = []

        def ship(k):
            scat(k)
            out_cps.append(
                pltpu.async_copy(
                    buf.at[:, pl.ds(k * t_chunk, t_chunk)],
                    out_hbm.at[b, :, pl.ds(t0 + k * t_chunk, t_chunk)],
                    sem_out,
                )
            )

        fill(0)
        id_cp.wait()
        for k in range(1, NW_CHUNKS):
            fill(k)
            ship(k - 1)
        ship(NW_CHUNKS - 1)
        for cp in out_cps:
            cp.wait()

    out_t = sc_onehot(ids)
    return jnp.swapaxes(out_t, 1, 2)
